# baseline (speedup 1.0000x reference)
import functools

import jax
import jax.numpy as jnp
from jax import lax
from jax.experimental import pallas as pl
from jax.experimental.pallas import tpu as pltpu
from jax.experimental.pallas import tpu_sc as plsc

EMBED = 64
SEQ = 200
BATCH = 4096
HALF = BATCH // 2
MID_W = 128
MID_ROWS = SEQ * HALF
NC, NS = 2, 16
NW = NC * NS
B2 = HALF // NW
SCALE = 8.0
NBUF = 4
SBLK = 5


def _sc_gather(idx_t, table):
  mesh = plsc.VectorSubcoreMesh(core_axis_name="c", subcore_axis_name="s")

  @functools.partial(
      pl.kernel,
      mesh=mesh,
      compiler_params=pltpu.CompilerParams(use_tc_tiling_on_sc=False),
      out_type=jax.ShapeDtypeStruct((MID_ROWS, MID_W), jnp.float32),
      scratch_types=[
          pltpu.VMEM((SEQ, B2), jnp.int32),
          pltpu.VMEM((SEQ, B2), jnp.int32),
          [pltpu.VMEM((B2, EMBED), jnp.float32)] * NBUF,
          [pltpu.VMEM((B2, EMBED), jnp.float32)] * NBUF,
          [pltpu.SemaphoreType.DMA] * NBUF,
          [pltpu.SemaphoreType.DMA] * NBUF,
      ],
  )
  def k(idx_hbm, table_hbm, mid_hbm, idx_lo, idx_hi, blo, bhi, gsem, ssem):
    wid = lax.axis_index("s") * NC + lax.axis_index("c")
    b0 = wid * B2
    pltpu.sync_copy(idx_hbm.at[:, pl.ds(b0, B2)], idx_lo)
    pltpu.sync_copy(idx_hbm.at[:, pl.ds(HALF + b0, B2)], idx_hi)

    def start_gather(s, b):
      pltpu.async_copy(table_hbm.at[idx_lo.at[s]], blo[b], gsem[b])
      pltpu.async_copy(table_hbm.at[idx_hi.at[s]], bhi[b], gsem[b])

    def wait_gather(b):
      pltpu.make_async_copy(table_hbm.at[idx_lo.at[0]], blo[b], gsem[b]).wait()
      pltpu.make_async_copy(table_hbm.at[idx_hi.at[0]], bhi[b], gsem[b]).wait()

    def start_scatter(s, b):
      r0 = s * HALF + b0
      pltpu.async_copy(
          blo[b], mid_hbm.at[pl.ds(r0, B2), pl.ds(0, EMBED)], ssem[b])
      pltpu.async_copy(
          bhi[b], mid_hbm.at[pl.ds(r0, B2), pl.ds(EMBED, EMBED)], ssem[b])

    def wait_scatter(b):
      pltpu.make_async_copy(
          blo[b], mid_hbm.at[pl.ds(0, B2), pl.ds(0, EMBED)], ssem[b]).wait()
      pltpu.make_async_copy(
          bhi[b], mid_hbm.at[pl.ds(0, B2), pl.ds(EMBED, EMBED)], ssem[b]).wait()

    start_gather(0, 0)
    start_gather(1, 1)

    def step(i, carry):
      for b in range(NBUF):
        s = i * NBUF + b
        wait_gather(b)
        nb = (b + 2) % NBUF

        @pl.when(s >= 2)
        def _():
          wait_scatter(nb)

        @pl.when(s + 2 < SEQ)
        def _():
          start_gather(s + 2, nb)

        start_scatter(s, b)
      return carry

    lax.fori_loop(0, SEQ // NBUF, step, 0)
    wait_scatter((SEQ - 2) % NBUF)
    wait_scatter((SEQ - 1) % NBUF)

  return k(idx_t, table)


def _tc_finish(mid3, pos):
  def body(in_ref, pos_ref, out_ref):
    r = lax.broadcasted_iota(jnp.int32, (EMBED, EMBED), 0)
    c = lax.broadcasted_iota(jnp.int32, (EMBED, EMBED), 1)
    eye = jnp.where(r == c, SCALE, 0.0).astype(jnp.float32)
    for i in range(SBLK):
      x = in_ref[i]
      p = pos_ref[pl.ds(pl.program_id(0) * SBLK + i, 1), :]
      for h in range(2):
        v = x[:, h * EMBED:(h + 1) * EMBED]
        y = lax.dot_general(eye, v, (((1,), (1,)), ((), ())),
                            preferred_element_type=jnp.float32)
        out_ref[i, :, pl.ds(h * HALF, HALF)] = y + p.T

  return pl.pallas_call(
      body,
      grid=(SEQ // SBLK,),
      in_specs=[
          pl.BlockSpec((SBLK, HALF, MID_W), lambda s: (s, 0, 0)),
          pl.BlockSpec((512, EMBED), lambda s: (0, 0)),
      ],
      out_specs=pl.BlockSpec((SBLK, EMBED, BATCH), lambda s: (s, 0, 0)),
      out_shape=jax.ShapeDtypeStruct((SEQ, EMBED, BATCH), jnp.float32),
  )(mid3, pos)


def kernel(input_tensor, src_table, pos_table):
  idx_t = input_tensor.T.astype(jnp.int32)
  mid = _sc_gather(idx_t, src_table)
  mid3 = mid.reshape(SEQ, HALF, MID_W)
  out_t = _tc_finish(mid3, pos_table)
  return jnp.transpose(out_t, (2, 0, 1))


# speedup vs baseline: 1.0199x; 1.0199x over previous
import functools

import jax
import jax.numpy as jnp
from jax import lax
from jax.experimental import pallas as pl
from jax.experimental.pallas import tpu as pltpu
from jax.experimental.pallas import tpu_sc as plsc

EMBED = 64
SEQ = 200
BATCH = 4096
HALF = BATCH // 2
MID_W = 128
MID_ROWS = SEQ * HALF
NC, NS = 2, 16
NW = NC * NS
B2 = HALF // NW
SCALE = 8.0
NBUF = 8
PREF = 4
SBLK = 5


def _sc_gather(idx_t, table):
  mesh = plsc.VectorSubcoreMesh(core_axis_name="c", subcore_axis_name="s")

  @functools.partial(
      pl.kernel,
      mesh=mesh,
      compiler_params=pltpu.CompilerParams(use_tc_tiling_on_sc=False),
      out_type=jax.ShapeDtypeStruct((MID_ROWS, MID_W), jnp.float32),
      scratch_types=[
          pltpu.VMEM((SEQ, B2), jnp.int32),
          pltpu.VMEM((SEQ, B2), jnp.int32),
          [pltpu.VMEM((B2, EMBED), jnp.float32)] * NBUF,
          [pltpu.VMEM((B2, EMBED), jnp.float32)] * NBUF,
          [pltpu.SemaphoreType.DMA] * NBUF,
          [pltpu.SemaphoreType.DMA] * NBUF,
      ],
  )
  def k(idx_hbm, table_hbm, mid_hbm, idx_lo, idx_hi, blo, bhi, gsem, ssem):
    wid = lax.axis_index("s") * NC + lax.axis_index("c")
    b0 = wid * B2
    pltpu.sync_copy(idx_hbm.at[:, pl.ds(b0, B2)], idx_lo)
    pltpu.sync_copy(idx_hbm.at[:, pl.ds(HALF + b0, B2)], idx_hi)

    def start_gather(s, b):
      pltpu.async_copy(table_hbm.at[idx_lo.at[s]], blo[b], gsem[b])
      pltpu.async_copy(table_hbm.at[idx_hi.at[s]], bhi[b], gsem[b])

    def wait_gather(b):
      pltpu.make_async_copy(table_hbm.at[idx_lo.at[0]], blo[b], gsem[b]).wait()
      pltpu.make_async_copy(table_hbm.at[idx_hi.at[0]], bhi[b], gsem[b]).wait()

    def start_scatter(s, b):
      r0 = s * HALF + b0
      pltpu.async_copy(
          blo[b], mid_hbm.at[pl.ds(r0, B2), pl.ds(0, EMBED)], ssem[b])
      pltpu.async_copy(
          bhi[b], mid_hbm.at[pl.ds(r0, B2), pl.ds(EMBED, EMBED)], ssem[b])

    def wait_scatter(b):
      pltpu.make_async_copy(
          blo[b], mid_hbm.at[pl.ds(0, B2), pl.ds(0, EMBED)], ssem[b]).wait()
      pltpu.make_async_copy(
          bhi[b], mid_hbm.at[pl.ds(0, B2), pl.ds(EMBED, EMBED)], ssem[b]).wait()

    for j in range(PREF):
      start_gather(j, j)

    def step(i, carry):
      for b in range(NBUF):
        s = i * NBUF + b
        wait_gather(b)
        nb = (b + PREF) % NBUF

        @pl.when(s >= PREF)
        def _():
          wait_scatter(nb)

        @pl.when(s + PREF < SEQ)
        def _():
          start_gather(s + PREF, nb)

        start_scatter(s, b)
      return carry

    lax.fori_loop(0, SEQ // NBUF, step, 0)
    for j in range(PREF):
      wait_scatter((SEQ - PREF + j) % NBUF)

  return k(idx_t, table)


def _tc_finish(mid3, pos):
  def body(in_ref, pos_ref, out_ref):
    r = lax.broadcasted_iota(jnp.int32, (EMBED, EMBED), 0)
    c = lax.broadcasted_iota(jnp.int32, (EMBED, EMBED), 1)
    eye = jnp.where(r == c, SCALE, 0.0).astype(jnp.float32)
    for i in range(SBLK):
      x = in_ref[i]
      p = pos_ref[pl.ds(pl.program_id(0) * SBLK + i, 1), :]
      for h in range(2):
        v = x[:, h * EMBED:(h + 1) * EMBED]
        y = lax.dot_general(eye, v, (((1,), (1,)), ((), ())),
                            preferred_element_type=jnp.float32)
        out_ref[i, :, pl.ds(h * HALF, HALF)] = y + p.T

  return pl.pallas_call(
      body,
      grid=(SEQ // SBLK,),
      in_specs=[
          pl.BlockSpec((SBLK, HALF, MID_W), lambda s: (s, 0, 0)),
          pl.BlockSpec((512, EMBED), lambda s: (0, 0)),
      ],
      out_specs=pl.BlockSpec((SBLK, EMBED, BATCH), lambda s: (s, 0, 0)),
      out_shape=jax.ShapeDtypeStruct((SEQ, EMBED, BATCH), jnp.float32),
  )(mid3, pos)


def kernel(input_tensor, src_table, pos_table):
  idx_t = input_tensor.T.astype(jnp.int32)
  mid = _sc_gather(idx_t, src_table)
  mid3 = mid.reshape(SEQ, HALF, MID_W)
  out_t = _tc_finish(mid3, pos_table)
  return jnp.transpose(out_t, (2, 0, 1))
